# Initial kernel scaffold; baseline (speedup 1.0000x reference)
import jax
import jax.numpy as jnp
from jax.experimental import pallas as pl


def kernel(y, x, t2v_w, t2v_b, local_table, vt_W, vt_b, given_table, space_table):
    raise NotImplementedError("write your pallas kernel here")


# speedup vs baseline: 7.8864x; 7.8864x over previous
import jax
import jax.numpy as jnp
from jax.experimental import pallas as pl
from jax.experimental.pallas import tpu as pltpu

_B, _L, _J, _D = 4, 2048, 8, 256
_DX, _TED = 6, 6
_T = _DX * _TED


def _emb_kernel(x_ref, y_ref, s_ref, bflat_ref, w1_ref, w0_ref, vtb_ref,
                local_ref, given_ref, space_ref,
                out_vt_ref, out_sp_ref, base_ref):
    j = pl.program_id(1)

    @pl.when(j == 0)
    def _():
        xb = jnp.nan_to_num(x_ref[0])
        xa = jax.lax.dot(xb, s_ref[...], precision=jax.lax.Precision.HIGHEST,
                         preferred_element_type=jnp.float32) + bflat_ref[...]
        lane = jax.lax.broadcasted_iota(jnp.int32, (_L, _T), 1)
        te = jnp.where(lane % _TED == 0, xa, jnp.sin(xa))
        base_ref[...] = (local_ref[...]
                         + jax.lax.dot(te, w1_ref[...],
                                       precision=jax.lax.Precision.HIGHEST,
                                       preferred_element_type=jnp.float32)
                         + vtb_ref[...])

    ycol = y_ref[0]
    nanmask = jnp.isnan(ycol)
    yc = jnp.where(nanmask, jnp.float32(0), ycol)
    gsel = jnp.where(nanmask, given_ref[0:1, :], given_ref[1:2, :])
    out_vt_ref[0] = base_ref[...] + yc * w0_ref[...] + gsel
    row = space_ref[pl.ds(j, 1), :]
    out_sp_ref[0] = jnp.broadcast_to(row, (_L, _D))


def kernel(y, x, t2v_w, t2v_b, local_table, vt_W, vt_b, given_table, space_table):
    s_mat = (jnp.eye(_DX, dtype=jnp.float32)[:, :, None]
             * t2v_w[:, None, :]).reshape(_DX, _T)
    b_flat = t2v_b.reshape(1, _T)
    w1 = vt_W[1:]
    w0 = vt_W[0:1]
    vtb = vt_b.reshape(1, _D)
    y_t = jnp.transpose(y, (0, 2, 1)).reshape(_B * _J, _L, 1)

    grid = (_B, _J)
    out_vt, out_sp = pl.pallas_call(
        _emb_kernel,
        grid=grid,
        in_specs=[
            pl.BlockSpec((1, _L, _DX), lambda b, j: (b, 0, 0)),
            pl.BlockSpec((1, _L, 1), lambda b, j: (b * _J + j, 0, 0)),
            pl.BlockSpec((_DX, _T), lambda b, j: (0, 0)),
            pl.BlockSpec((1, _T), lambda b, j: (0, 0)),
            pl.BlockSpec((_T, _D), lambda b, j: (0, 0)),
            pl.BlockSpec((1, _D), lambda b, j: (0, 0)),
            pl.BlockSpec((1, _D), lambda b, j: (0, 0)),
            pl.BlockSpec((_L, _D), lambda b, j: (0, 0)),
            pl.BlockSpec((2, _D), lambda b, j: (0, 0)),
            pl.BlockSpec((_J, _D), lambda b, j: (0, 0)),
        ],
        out_specs=[
            pl.BlockSpec((1, _L, _D), lambda b, j: (b, j, 0)),
            pl.BlockSpec((1, _L, _D), lambda b, j: (b, j, 0)),
        ],
        out_shape=[
            jax.ShapeDtypeStruct((_B, _J * _L, _D), jnp.float32),
            jax.ShapeDtypeStruct((_B, _J * _L, _D), jnp.float32),
        ],
        scratch_shapes=[pltpu.VMEM((_L, _D), jnp.float32)],
    )(x, y_t, s_mat, b_flat, w1, w0, vtb,
      local_table[:_L], given_table, space_table)

    var_idx = jnp.broadcast_to(
        jnp.repeat(jnp.arange(_J, dtype=jnp.int32), _L)[None, :],
        (_B, _J * _L))
    return (out_vt, out_sp, var_idx)


# speedup vs baseline: 7.9049x; 1.0023x over previous
import jax
import jax.numpy as jnp
from jax.experimental import pallas as pl
from jax.experimental.pallas import tpu as pltpu

_B, _L, _J, _D = 4, 2048, 8, 256
_DX, _TED = 6, 6
_T = _DX * _TED


def _emb_kernel(x_ref, y_ref, s_ref, bflat_ref, w1_ref, w0_ref, vtb_ref,
                local_ref, given_ref, space_ref,
                out_vt_ref, out_sp_ref, base_ref):
    j = pl.program_id(1)

    @pl.when(j == 0)
    def _():
        xb = jnp.nan_to_num(x_ref[0])
        xa = jax.lax.dot(xb, s_ref[...], precision=jax.lax.Precision.HIGHEST,
                         preferred_element_type=jnp.float32) + bflat_ref[...]
        lane = jax.lax.broadcasted_iota(jnp.int32, (_L, _T), 1)
        te = jnp.where(lane % _TED == 0, xa, jnp.sin(xa))
        base_ref[...] = (local_ref[...]
                         + jax.lax.dot(te, w1_ref[...],
                                       precision=jax.lax.Precision.HIGHEST,
                                       preferred_element_type=jnp.float32)
                         + vtb_ref[...])

    ycol = y_ref[0]
    nanmask = jnp.isnan(ycol)
    yc = jnp.where(nanmask, jnp.float32(0), ycol)
    gsel = jnp.where(nanmask, given_ref[0:1, :], given_ref[1:2, :])
    out_vt_ref[0] = base_ref[...] + yc * w0_ref[...] + gsel
    row = space_ref[pl.ds(j, 1), :]
    out_sp_ref[0] = jnp.broadcast_to(row, (_L, _D))


def kernel(y, x, t2v_w, t2v_b, local_table, vt_W, vt_b, given_table, space_table):
    s_mat = (jnp.eye(_DX, dtype=jnp.float32)[:, :, None]
             * t2v_w[:, None, :]).reshape(_DX, _T)
    b_flat = t2v_b.reshape(1, _T)
    w1 = vt_W[1:]
    w0 = vt_W[0:1]
    vtb = vt_b.reshape(1, _D)
    y_t = jnp.transpose(y, (0, 2, 1)).reshape(_B * _J, _L, 1)

    grid = (_B, _J)
    out_vt, out_sp = pl.pallas_call(
        _emb_kernel,
        grid=grid,
        in_specs=[
            pl.BlockSpec((1, _L, _DX), lambda b, j: (b, 0, 0)),
            pl.BlockSpec((1, _L, 1), lambda b, j: (b * _J + j, 0, 0)),
            pl.BlockSpec((_DX, _T), lambda b, j: (0, 0)),
            pl.BlockSpec((1, _T), lambda b, j: (0, 0)),
            pl.BlockSpec((_T, _D), lambda b, j: (0, 0)),
            pl.BlockSpec((1, _D), lambda b, j: (0, 0)),
            pl.BlockSpec((1, _D), lambda b, j: (0, 0)),
            pl.BlockSpec((_L, _D), lambda b, j: (0, 0)),
            pl.BlockSpec((2, _D), lambda b, j: (0, 0)),
            pl.BlockSpec((_J, _D), lambda b, j: (0, 0)),
        ],
        out_specs=[
            pl.BlockSpec((1, _L, _D), lambda b, j: (b, j, 0)),
            pl.BlockSpec((1, _L, _D), lambda b, j: (b, j, 0)),
        ],
        out_shape=[
            jax.ShapeDtypeStruct((_B, _J * _L, _D), jnp.float32),
            jax.ShapeDtypeStruct((_B, _J * _L, _D), jnp.float32),
        ],
        scratch_shapes=[pltpu.VMEM((_L, _D), jnp.float32)],
        compiler_params=pltpu.CompilerParams(
            dimension_semantics=("parallel", "arbitrary")),
    )(x, y_t, s_mat, b_flat, w1, w0, vtb,
      local_table[:_L], given_table, space_table)

    var_idx = jnp.broadcast_to(
        jnp.repeat(jnp.arange(_J, dtype=jnp.int32), _L)[None, :],
        (_B, _J * _L))
    return (out_vt, out_sp, var_idx)
